# baseline (speedup 1.0000x reference)
import functools

import jax
import jax.numpy as jnp
from jax import lax
from jax.experimental import pallas as pl
from jax.experimental.pallas import tpu as pltpu
from jax.experimental.pallas import tpu_sc as plsc

_LANES = 16


def _sc_mesh():
    return plsc.VectorSubcoreMesh(core_axis_name="c", subcore_axis_name="s")


def _build_hist(n, nw, ns, nit):
    @functools.partial(
        pl.kernel,
        out_type=jax.ShapeDtypeStruct((nw, n), jnp.float32),
        mesh=_sc_mesh(),
        scratch_types=[
            pltpu.VMEM((nit, _LANES), jnp.int32),
            pltpu.VMEM((n,), jnp.float32),
        ],
        compiler_params=pltpu.CompilerParams(needs_layout_passes=False),
    )
    def hist(dst_hbm, out_hbm, idx_v, hist_v):
        cid = lax.axis_index("c")
        sid = lax.axis_index("s")
        wid = cid * ns + sid
        pltpu.sync_copy(dst_hbm.at[wid], idx_v)

        zero16 = jnp.zeros((_LANES,), jnp.float32)

        def zbody(i, carry):
            hist_v[pl.ds(i * _LANES, _LANES)] = zero16
            return carry

        lax.fori_loop(0, n // _LANES, zbody, 0)

        ones16 = jnp.ones((_LANES,), jnp.float32)

        def body(i, carry):
            idx = idx_v[i, :]
            plsc.addupdate_scatter(hist_v, [idx], ones16)
            return carry

        lax.fori_loop(0, nit, body, 0)
        pltpu.sync_copy(hist_v, out_hbm.at[wid])

    return hist


def _build_scatter(n_pad, f, nc, ns, nh, ch, k):
    rows_per_tile = n_pad // ns
    zrows = 64

    @functools.partial(
        pl.kernel,
        out_type=jax.ShapeDtypeStruct((nc, n_pad, f), jnp.float32),
        mesh=_sc_mesh(),
        scratch_types=[
            pltpu.VMEM((ch, k), jnp.int32),
            pltpu.VMEM((ch, k), jnp.int32),
            pltpu.VMEM((k, f), jnp.float32),
            pltpu.VMEM((k, f), jnp.float32),
            pltpu.VMEM_SHARED((n_pad, f), jnp.float32),
            pltpu.SemaphoreType.DMA,
            pltpu.SemaphoreType.DMA,
            pltpu.SemaphoreType.DMA,
            pltpu.SemaphoreType.DMA,
        ],
    )
    def scat(y_hbm, src_hbm, dst_hbm, out_hbm, src_v, dst_v, rows0, rows1,
             agg_sh, sg0, sg1, ss0, ss1):
        cid = lax.axis_index("c")
        sid = lax.axis_index("s")
        wid = cid * ns + sid

        zero16 = jnp.zeros((_LANES,), jnp.float32)
        fparts = f // _LANES

        def zbody(i, carry):
            r = i // fparts
            c = i % fparts
            rows0[r, pl.ds(c * _LANES, _LANES)] = zero16
            return carry

        lax.fori_loop(0, zrows * fparts, zbody, 0)

        base = sid * rows_per_tile
        zsrc = rows0.at[pl.ds(0, zrows)]
        for j in range(rows_per_tile // zrows):
            pltpu.sync_copy(zsrc, agg_sh.at[pl.ds(base + j * zrows, zrows)])
        plsc.subcore_barrier()

        def g(j, buf, sem):
            pltpu.async_copy(y_hbm.at[src_v.at[j]], buf, sem)

        def gwait(j, buf, sem):
            pltpu.make_async_copy(y_hbm.at[src_v.at[j]], buf, sem).wait()

        def s(j, buf, sem):
            pltpu.async_copy(buf, agg_sh.at[dst_v.at[j]], sem, add=True)

        def swait(j, buf, sem):
            pltpu.make_async_copy(buf, agg_sh.at[dst_v.at[j]], sem).wait()

        for hh in range(nh):
            pltpu.sync_copy(src_hbm.at[wid, hh], src_v)
            pltpu.sync_copy(dst_hbm.at[wid, hh], dst_v)

            g(0, rows0, sg0)
            gwait(0, rows0, sg0)
            s(0, rows0, ss0)
            g(1, rows1, sg1)
            gwait(1, rows1, sg1)
            s(1, rows1, ss1)
            swait(0, rows0, ss0)
            g(2, rows0, sg0)

            def pair(jj, carry):
                j0 = 2 * jj + 2
                gwait(j0, rows0, sg0)
                s(j0, rows0, ss0)
                swait(j0 - 1, rows1, ss1)
                g(j0 + 1, rows1, sg1)
                gwait(j0 + 1, rows1, sg1)
                s(j0 + 1, rows1, ss1)
                swait(j0, rows0, ss0)
                g(j0 + 2, rows0, sg0)
                return carry

            lax.fori_loop(0, ch // 2 - 2, pair, 0)

            j0 = ch - 2
            gwait(j0, rows0, sg0)
            s(j0, rows0, ss0)
            swait(j0 - 1, rows1, ss1)
            g(j0 + 1, rows1, sg1)
            gwait(j0 + 1, rows1, sg1)
            s(j0 + 1, rows1, ss1)
            swait(j0, rows0, ss0)
            swait(j0 + 1, rows1, ss1)

        plsc.subcore_barrier()
        pltpu.sync_copy(
            agg_sh.at[pl.ds(base, rows_per_tile)],
            out_hbm.at[cid, pl.ds(base, rows_per_tile)],
        )

    return scat


def _b1_body(f_ref, w_ref, x_ref):
    x_ref[...] = jnp.dot(f_ref[...], w_ref[...],
                         preferred_element_type=jnp.float32)


def _b2_body(deg_ref, x_ref, y_ref):
    deg = jnp.sum(deg_ref[...], axis=0) + 1.0
    dinv = lax.rsqrt(deg)
    y_ref[...] = x_ref[...] * dinv[:, None]


def _d_body(s_ref, y_ref, deg_ref, b_ref, xo_ref, h_ref):
    n = y_ref.shape[0]
    deg = jnp.sum(deg_ref[...], axis=0) + 1.0
    dinv = lax.rsqrt(deg)
    tot = (s_ref[0, :n, :] + s_ref[1, :n, :] + y_ref[...]) * dinv[:, None] \
        + b_ref[...][None, :]
    xo = jnp.maximum(tot, 0.0)
    xo_ref[...] = xo
    h_ref[...] = jax.nn.sigmoid(jnp.mean(xo, axis=0, keepdims=True))


def kernel(feature, edge_index, W, b):
    n, f_in = feature.shape
    f_out = W.shape[1]
    e = edge_index.shape[1]

    info = plsc.get_sparse_core_info()
    nc, ns = info.num_cores, info.num_subcores
    nw = nc * ns
    epw = e // nw
    k = 100
    nh = 2
    ch = epw // (k * nh)
    align = ns * 128
    n_pad = (n + align - 1) // align * align
    assert epw * nw == e and nh * ch * k == epw and ch % 2 == 0 and ch >= 4
    assert f_out % _LANES == 0

    src3 = edge_index[0].reshape(nw, nh, ch, k)
    dst3 = edge_index[1].reshape(nw, nh, ch, k)
    nit = epw // _LANES
    dst_h = edge_index[1].reshape(nw, nit, _LANES)

    deg_parts = _build_hist(n, nw, ns, nit)(dst_h)

    x = pl.pallas_call(
        _b1_body,
        out_shape=jax.ShapeDtypeStruct((n, f_out), jnp.float32),
    )(feature, W)

    y = pl.pallas_call(
        _b2_body,
        out_shape=jax.ShapeDtypeStruct((n, f_out), jnp.float32),
    )(deg_parts, x)

    s_parts = _build_scatter(n_pad, f_out, nc, ns, nh, ch, k)(y, src3, dst3)

    x_out, h = pl.pallas_call(
        _d_body,
        out_shape=(
            jax.ShapeDtypeStruct((n, f_out), jnp.float32),
            jax.ShapeDtypeStruct((1, f_out), jnp.float32),
        ),
    )(s_parts, y, deg_parts, b)
    return (x_out, h.reshape(f_out))


# speedup vs baseline: 1.0206x; 1.0206x over previous
import functools

import jax
import jax.numpy as jnp
from jax import lax
from jax.experimental import pallas as pl
from jax.experimental.pallas import tpu as pltpu
from jax.experimental.pallas import tpu_sc as plsc

_LANES = 16


def _sc_mesh():
    return plsc.VectorSubcoreMesh(core_axis_name="c", subcore_axis_name="s")


def _build_hist(n, nw, ns, nit):
    @functools.partial(
        pl.kernel,
        out_type=jax.ShapeDtypeStruct((nw, n), jnp.float32),
        mesh=_sc_mesh(),
        scratch_types=[
            pltpu.VMEM((nit, _LANES), jnp.int32),
            pltpu.VMEM((n,), jnp.float32),
        ],
        compiler_params=pltpu.CompilerParams(needs_layout_passes=False),
    )
    def hist(dst_hbm, out_hbm, idx_v, hist_v):
        cid = lax.axis_index("c")
        sid = lax.axis_index("s")
        wid = cid * ns + sid
        pltpu.sync_copy(dst_hbm.at[wid], idx_v)

        zero16 = jnp.zeros((_LANES,), jnp.float32)

        def zbody(i, carry):
            hist_v[pl.ds(i * _LANES, _LANES)] = zero16
            return carry

        lax.fori_loop(0, n // _LANES, zbody, 0)

        ones16 = jnp.ones((_LANES,), jnp.float32)

        def body(i, carry):
            idx = idx_v[i, :]
            plsc.addupdate_scatter(hist_v, [idx], ones16)
            return carry

        lax.fori_loop(0, nit, body, 0)
        pltpu.sync_copy(hist_v, out_hbm.at[wid])

    return hist


def _build_scatter(n_pad, f, nc, ns, ch, k):
    rows_per_tile = n_pad // ns
    zrows = 128
    qch = 16
    nq = ch // qch

    @functools.partial(
        pl.kernel,
        out_type=jax.ShapeDtypeStruct((nc, n_pad, f), jnp.float32),
        mesh=_sc_mesh(),
        scratch_types=[
            pltpu.VMEM((qch, k), jnp.int32),
            pltpu.VMEM((qch, k), jnp.int32),
            pltpu.VMEM((zrows, f), jnp.float32),
            pltpu.VMEM((k, f), jnp.float32),
            pltpu.VMEM_SHARED((n_pad, f), jnp.float32),
            pltpu.SemaphoreType.DMA,
            pltpu.SemaphoreType.DMA,
        ],
    )
    def scat(y_hbm, src_hbm, dst_hbm, out_hbm, src_v, dst_v, rows0, rows1,
             agg_sh, sem0, sem1):
        cid = lax.axis_index("c")
        sid = lax.axis_index("s")
        wid = cid * ns + sid

        zero16 = jnp.zeros((_LANES,), jnp.float32)
        fparts = f // _LANES

        def zbody(i, carry):
            r = i // fparts
            c = i % fparts
            rows0[r, pl.ds(c * _LANES, _LANES)] = zero16
            return carry

        lax.fori_loop(0, zrows * fparts, zbody, 0)

        base = sid * rows_per_tile
        for j in range(rows_per_tile // zrows):
            pltpu.sync_copy(rows0, agg_sh.at[pl.ds(base + j * zrows, zrows)])
        plsc.subcore_barrier()

        rows0k = rows0.at[pl.ds(0, k)]

        for q in range(nq):
            pltpu.sync_copy(src_hbm.at[wid, pl.ds(q * qch, qch)], src_v)
            pltpu.sync_copy(dst_hbm.at[wid, pl.ds(q * qch, qch)], dst_v)
            pltpu.async_copy(y_hbm.at[src_v.at[0]], rows0k, sem0)

            def pair(jj, carry):
                j0 = 2 * jj
                pltpu.make_async_copy(y_hbm.at[src_v.at[j0]], rows0k,
                                      sem0).wait()
                pltpu.async_copy(y_hbm.at[src_v.at[j0 + 1]], rows1, sem1)
                pltpu.sync_copy(rows0k, agg_sh.at[dst_v.at[j0]], add=True)
                pltpu.make_async_copy(y_hbm.at[src_v.at[j0 + 1]], rows1,
                                      sem1).wait()
                pltpu.async_copy(y_hbm.at[src_v.at[j0 + 2]], rows0k, sem0)
                pltpu.sync_copy(rows1, agg_sh.at[dst_v.at[j0 + 1]], add=True)
                return carry

            lax.fori_loop(0, qch // 2 - 1, pair, 0)

            j0 = qch - 2
            pltpu.make_async_copy(y_hbm.at[src_v.at[j0]], rows0k, sem0).wait()
            pltpu.async_copy(y_hbm.at[src_v.at[j0 + 1]], rows1, sem1)
            pltpu.sync_copy(rows0k, agg_sh.at[dst_v.at[j0]], add=True)
            pltpu.make_async_copy(y_hbm.at[src_v.at[j0 + 1]], rows1,
                                  sem1).wait()
            pltpu.sync_copy(rows1, agg_sh.at[dst_v.at[j0 + 1]], add=True)

        plsc.subcore_barrier()
        pltpu.sync_copy(
            agg_sh.at[pl.ds(base, rows_per_tile)],
            out_hbm.at[cid, pl.ds(base, rows_per_tile)],
        )

    return scat


def _b1_body(f_ref, w_ref, x_ref):
    x_ref[...] = jnp.dot(f_ref[...], w_ref[...],
                         preferred_element_type=jnp.float32)


def _b2_body(deg_ref, x_ref, y_ref):
    deg = jnp.sum(deg_ref[...], axis=0) + 1.0
    dinv = lax.rsqrt(deg)
    y_ref[...] = x_ref[...] * dinv[:, None]


def _d_body(s_ref, y_ref, deg_ref, b_ref, xo_ref, h_ref):
    n = y_ref.shape[0]
    deg = jnp.sum(deg_ref[...], axis=0) + 1.0
    dinv = lax.rsqrt(deg)
    tot = (s_ref[0, :n, :] + s_ref[1, :n, :] + y_ref[...]) * dinv[:, None] \
        + b_ref[...][None, :]
    xo = jnp.maximum(tot, 0.0)
    xo_ref[...] = xo
    h_ref[...] = jax.nn.sigmoid(jnp.mean(xo, axis=0, keepdims=True))


def kernel(feature, edge_index, W, b):
    n, f_in = feature.shape
    f_out = W.shape[1]
    e = edge_index.shape[1]

    info = plsc.get_sparse_core_info()
    nc, ns = info.num_cores, info.num_subcores
    nw = nc * ns
    epw = e // nw
    k = 125
    ch = epw // k
    align = ns * 128
    n_pad = (n + align - 1) // align * align
    assert epw * nw == e and ch * k == epw and ch % 16 == 0
    assert f_out % _LANES == 0

    src3 = edge_index[0].reshape(nw, ch, k)
    dst3 = edge_index[1].reshape(nw, ch, k)
    nit = epw // _LANES
    dst_h = edge_index[1].reshape(nw, nit, _LANES)

    deg_parts = _build_hist(n, nw, ns, nit)(dst_h)

    x = pl.pallas_call(
        _b1_body,
        out_shape=jax.ShapeDtypeStruct((n, f_out), jnp.float32),
    )(feature, W)

    y = pl.pallas_call(
        _b2_body,
        out_shape=jax.ShapeDtypeStruct((n, f_out), jnp.float32),
    )(deg_parts, x)

    s_parts = _build_scatter(n_pad, f_out, nc, ns, ch, k)(y, src3, dst3)

    x_out, h = pl.pallas_call(
        _d_body,
        out_shape=(
            jax.ShapeDtypeStruct((n, f_out), jnp.float32),
            jax.ShapeDtypeStruct((1, f_out), jnp.float32),
        ),
    )(s_parts, y, deg_parts, b)
    return (x_out, h.reshape(f_out))


# speedup vs baseline: 1.0269x; 1.0062x over previous
import functools

import jax
import jax.numpy as jnp
from jax import lax
from jax.experimental import pallas as pl
from jax.experimental.pallas import tpu as pltpu
from jax.experimental.pallas import tpu_sc as plsc

_LANES = 16


def _sc_mesh():
    return plsc.VectorSubcoreMesh(core_axis_name="c", subcore_axis_name="s")


def _build_hist(n_pad, nw, ns, nit, unroll):
    @functools.partial(
        pl.kernel,
        out_type=jax.ShapeDtypeStruct((nw, n_pad), jnp.float32),
        mesh=_sc_mesh(),
        scratch_types=[
            pltpu.VMEM((nit, _LANES), jnp.int32),
            pltpu.VMEM((n_pad,), jnp.float32),
        ],
        compiler_params=pltpu.CompilerParams(needs_layout_passes=False),
    )
    def hist(dst_hbm, out_hbm, idx_v, hist_v):
        cid = lax.axis_index("c")
        sid = lax.axis_index("s")
        wid = cid * ns + sid
        pltpu.sync_copy(dst_hbm.at[wid], idx_v)

        zero16 = jnp.zeros((_LANES,), jnp.float32)
        nz = n_pad // _LANES

        def zbody(i, carry):
            for jj in range(unroll):
                hist_v[pl.ds((i * unroll + jj) * _LANES, _LANES)] = zero16
            return carry

        lax.fori_loop(0, nz // unroll, zbody, 0)

        ones16 = jnp.ones((_LANES,), jnp.float32)

        def body(i, carry):
            for jj in range(unroll):
                idx = idx_v[i * unroll + jj, :]
                plsc.addupdate_scatter(hist_v, [idx], ones16)
            return carry

        lax.fori_loop(0, nit // unroll, body, 0)
        pltpu.sync_copy(hist_v, out_hbm.at[wid])

    return hist


def _build_scatter(n_pad, f, nc, ns, ch, k):
    rows_per_tile = n_pad // ns
    zrows = 128
    qch = 16
    nq = ch // qch

    @functools.partial(
        pl.kernel,
        out_type=jax.ShapeDtypeStruct((nc, n_pad, f), jnp.float32),
        mesh=_sc_mesh(),
        scratch_types=[
            pltpu.VMEM((qch, k), jnp.int32),
            pltpu.VMEM((qch, k), jnp.int32),
            pltpu.VMEM((zrows, f), jnp.float32),
            pltpu.VMEM((k, f), jnp.float32),
            pltpu.VMEM_SHARED((n_pad, f), jnp.float32),
            pltpu.SemaphoreType.DMA,
            pltpu.SemaphoreType.DMA,
        ],
    )
    def scat(y_hbm, src_hbm, dst_hbm, out_hbm, src_v, dst_v, rows0, rows1,
             agg_sh, sem0, sem1):
        cid = lax.axis_index("c")
        sid = lax.axis_index("s")
        wid = cid * ns + sid

        zero16 = jnp.zeros((_LANES,), jnp.float32)
        fparts = f // _LANES

        def zbody(i, carry):
            r = i // fparts
            c = i % fparts
            rows0[r, pl.ds(c * _LANES, _LANES)] = zero16
            return carry

        lax.fori_loop(0, zrows * fparts, zbody, 0)

        base = sid * rows_per_tile
        for j in range(rows_per_tile // zrows):
            pltpu.sync_copy(rows0, agg_sh.at[pl.ds(base + j * zrows, zrows)])
        plsc.subcore_barrier()

        rows0k = rows0.at[pl.ds(0, k)]

        for q in range(nq):
            pltpu.sync_copy(src_hbm.at[wid, pl.ds(q * qch, qch)], src_v)
            pltpu.sync_copy(dst_hbm.at[wid, pl.ds(q * qch, qch)], dst_v)
            pltpu.async_copy(y_hbm.at[src_v.at[0]], rows0k, sem0)

            def pair(jj, carry):
                j0 = 2 * jj
                pltpu.make_async_copy(y_hbm.at[src_v.at[j0]], rows0k,
                                      sem0).wait()
                pltpu.async_copy(y_hbm.at[src_v.at[j0 + 1]], rows1, sem1)
                pltpu.sync_copy(rows0k, agg_sh.at[dst_v.at[j0]], add=True)
                pltpu.make_async_copy(y_hbm.at[src_v.at[j0 + 1]], rows1,
                                      sem1).wait()
                pltpu.async_copy(y_hbm.at[src_v.at[j0 + 2]], rows0k, sem0)
                pltpu.sync_copy(rows1, agg_sh.at[dst_v.at[j0 + 1]], add=True)
                return carry

            lax.fori_loop(0, qch // 2 - 1, pair, 0)

            j0 = qch - 2
            pltpu.make_async_copy(y_hbm.at[src_v.at[j0]], rows0k, sem0).wait()
            pltpu.async_copy(y_hbm.at[src_v.at[j0 + 1]], rows1, sem1)
            pltpu.sync_copy(rows0k, agg_sh.at[dst_v.at[j0]], add=True)
            pltpu.make_async_copy(y_hbm.at[src_v.at[j0 + 1]], rows1,
                                  sem1).wait()
            pltpu.sync_copy(rows1, agg_sh.at[dst_v.at[j0 + 1]], add=True)

        plsc.subcore_barrier()
        pltpu.sync_copy(
            agg_sh.at[pl.ds(base, rows_per_tile)],
            out_hbm.at[cid, pl.ds(base, rows_per_tile)],
        )

    return scat


def _b_body(deg_ref, f_ref, w_ref, y_ref):
    deg = jnp.sum(deg_ref[...], axis=0) + 1.0
    dinv = lax.rsqrt(deg)
    x = jnp.dot(f_ref[...], w_ref[...], preferred_element_type=jnp.float32)
    y_ref[...] = x * dinv[:, None]


def _d_body(n, s_ref, y_ref, deg_ref, b_ref, xo_ref, h_ref, acc_ref):
    i = pl.program_id(0)
    r = y_ref.shape[0]
    deg = jnp.sum(deg_ref[...], axis=0) + 1.0
    dinv = lax.rsqrt(deg)
    stot = jnp.sum(s_ref[...], axis=0)
    tot = (stot + y_ref[...]) * dinv[:, None] + b_ref[...]
    xo = jnp.maximum(tot, 0.0)
    xo_ref[...] = xo
    rows = lax.broadcasted_iota(jnp.int32, (r, 1), 0) + i * r
    part = jnp.sum(jnp.where(rows < n, xo, 0.0), axis=0, keepdims=True)

    @pl.when(i == 0)
    def _():
        acc_ref[...] = jnp.zeros_like(acc_ref)

    acc_ref[...] = acc_ref[...] + part
    h_ref[...] = jax.nn.sigmoid(acc_ref[...] * (1.0 / n))


def kernel(feature, edge_index, W, b):
    n, f_in = feature.shape
    f_out = W.shape[1]
    e = edge_index.shape[1]

    info = plsc.get_sparse_core_info()
    nc, ns = info.num_cores, info.num_subcores
    nw = nc * ns
    epw = e // nw
    k = 125
    ch = epw // k
    align = ns * 128
    n_pad = (n + align - 1) // align * align
    assert epw * nw == e and ch * k == epw and ch % 16 == 0
    assert f_out % _LANES == 0

    src3 = edge_index[0].reshape(nw, ch, k)
    dst3 = edge_index[1].reshape(nw, ch, k)
    nit = epw // _LANES
    dst_h = edge_index[1].reshape(nw, nit, _LANES)

    deg_parts = _build_hist(n_pad, nw, ns, nit, 5)(dst_h)

    rblk = 2048
    nsteps = (n + rblk - 1) // rblk
    b2 = b.reshape(1, f_out)

    y = pl.pallas_call(
        _b_body,
        grid=(nsteps,),
        in_specs=[
            pl.BlockSpec((nw, rblk), lambda i: (0, i)),
            pl.BlockSpec((rblk, f_in), lambda i: (i, 0)),
            pl.BlockSpec((f_in, f_out), lambda i: (0, 0)),
        ],
        out_specs=pl.BlockSpec((rblk, f_out), lambda i: (i, 0)),
        out_shape=jax.ShapeDtypeStruct((n, f_out), jnp.float32),
    )(deg_parts, feature, W)

    s_parts = _build_scatter(n_pad, f_out, nc, ns, ch, k)(y, src3, dst3)

    x_out, h = pl.pallas_call(
        functools.partial(_d_body, n),
        grid=(nsteps,),
        in_specs=[
            pl.BlockSpec((2, rblk, f_out), lambda i: (0, i, 0)),
            pl.BlockSpec((rblk, f_out), lambda i: (i, 0)),
            pl.BlockSpec((nw, rblk), lambda i: (0, i)),
            pl.BlockSpec((1, f_out), lambda i: (0, 0)),
        ],
        out_specs=(
            pl.BlockSpec((rblk, f_out), lambda i: (i, 0)),
            pl.BlockSpec((1, f_out), lambda i: (0, 0)),
        ),
        out_shape=(
            jax.ShapeDtypeStruct((n, f_out), jnp.float32),
            jax.ShapeDtypeStruct((1, f_out), jnp.float32),
        ),
        scratch_shapes=[pltpu.VMEM((1, f_out), jnp.float32)],
    )(s_parts, y, deg_parts, b2)
    return (x_out, h.reshape(f_out))


# speedup vs baseline: 1.0297x; 1.0028x over previous
import functools

import jax
import jax.numpy as jnp
from jax import lax
from jax.experimental import pallas as pl
from jax.experimental.pallas import tpu as pltpu
from jax.experimental.pallas import tpu_sc as plsc

_LANES = 16


def _sc_mesh():
    return plsc.VectorSubcoreMesh(core_axis_name="c", subcore_axis_name="s")


def _build_hist(n, nw, ns, nit):
    @functools.partial(
        pl.kernel,
        out_type=jax.ShapeDtypeStruct((nw, n), jnp.float32),
        mesh=_sc_mesh(),
        scratch_types=[
            pltpu.VMEM((nit, _LANES), jnp.int32),
            pltpu.VMEM((n,), jnp.float32),
        ],
        compiler_params=pltpu.CompilerParams(needs_layout_passes=False),
    )
    def hist(dst_hbm, out_hbm, idx_v, hist_v):
        cid = lax.axis_index("c")
        sid = lax.axis_index("s")
        wid = cid * ns + sid
        pltpu.sync_copy(dst_hbm.at[wid], idx_v)

        zero16 = jnp.zeros((_LANES,), jnp.float32)

        def zbody(i, carry):
            hist_v[pl.ds(i * _LANES, _LANES)] = zero16
            return carry

        lax.fori_loop(0, n // _LANES, zbody, 0)

        ones16 = jnp.ones((_LANES,), jnp.float32)

        def body(i, carry):
            idx = idx_v[i, :]
            plsc.addupdate_scatter(hist_v, [idx], ones16)
            return carry

        lax.fori_loop(0, nit, body, 0)
        pltpu.sync_copy(hist_v, out_hbm.at[wid])

    return hist


def _build_scatter(n_pad, f, nc, ns, ch, k):
    rows_per_tile = n_pad // ns
    zrows = 128
    qch = 16
    nq = ch // qch

    @functools.partial(
        pl.kernel,
        out_type=jax.ShapeDtypeStruct((nc, n_pad, f), jnp.float32),
        mesh=_sc_mesh(),
        scratch_types=[
            pltpu.VMEM((qch, k), jnp.int32),
            pltpu.VMEM((qch, k), jnp.int32),
            pltpu.VMEM((zrows, f), jnp.float32),
            pltpu.VMEM((k, f), jnp.float32),
            pltpu.VMEM_SHARED((n_pad, f), jnp.float32),
            pltpu.SemaphoreType.DMA,
            pltpu.SemaphoreType.DMA,
        ],
    )
    def scat(y_hbm, src_hbm, dst_hbm, out_hbm, src_v, dst_v, rows0, rows1,
             agg_sh, sem0, sem1):
        cid = lax.axis_index("c")
        sid = lax.axis_index("s")
        wid = cid * ns + sid

        zero16 = jnp.zeros((_LANES,), jnp.float32)
        fparts = f // _LANES

        def zbody(i, carry):
            r = i // fparts
            c = i % fparts
            rows0[r, pl.ds(c * _LANES, _LANES)] = zero16
            return carry

        lax.fori_loop(0, zrows * fparts, zbody, 0)

        base = sid * rows_per_tile
        for j in range(rows_per_tile // zrows):
            pltpu.sync_copy(rows0, agg_sh.at[pl.ds(base + j * zrows, zrows)])
        plsc.subcore_barrier()

        rows0k = rows0.at[pl.ds(0, k)]

        for q in range(nq):
            pltpu.sync_copy(src_hbm.at[wid, pl.ds(q * qch, qch)], src_v)
            pltpu.sync_copy(dst_hbm.at[wid, pl.ds(q * qch, qch)], dst_v)
            pltpu.async_copy(y_hbm.at[src_v.at[0]], rows0k, sem0)

            def pair(jj, carry):
                j0 = 2 * jj
                pltpu.make_async_copy(y_hbm.at[src_v.at[j0]], rows0k,
                                      sem0).wait()
                pltpu.async_copy(y_hbm.at[src_v.at[j0 + 1]], rows1, sem1)
                pltpu.sync_copy(rows0k, agg_sh.at[dst_v.at[j0]], add=True)
                pltpu.make_async_copy(y_hbm.at[src_v.at[j0 + 1]], rows1,
                                      sem1).wait()
                pltpu.async_copy(y_hbm.at[src_v.at[j0 + 2]], rows0k, sem0)
                pltpu.sync_copy(rows1, agg_sh.at[dst_v.at[j0 + 1]], add=True)
                return carry

            lax.fori_loop(0, qch // 2 - 1, pair, 0)

            j0 = qch - 2
            pltpu.make_async_copy(y_hbm.at[src_v.at[j0]], rows0k, sem0).wait()
            pltpu.async_copy(y_hbm.at[src_v.at[j0 + 1]], rows1, sem1)
            pltpu.sync_copy(rows0k, agg_sh.at[dst_v.at[j0]], add=True)
            pltpu.make_async_copy(y_hbm.at[src_v.at[j0 + 1]], rows1,
                                  sem1).wait()
            pltpu.sync_copy(rows1, agg_sh.at[dst_v.at[j0 + 1]], add=True)

        plsc.subcore_barrier()
        pltpu.sync_copy(
            agg_sh.at[pl.ds(base, rows_per_tile)],
            out_hbm.at[cid, pl.ds(base, rows_per_tile)],
        )

    return scat


def _b_body(deg_ref, f_ref, w_ref, y_ref):
    deg = jnp.sum(deg_ref[...], axis=0) + 1.0
    dinv = lax.rsqrt(deg)
    x = jnp.dot(f_ref[...], w_ref[...], preferred_element_type=jnp.float32)
    y_ref[...] = x * dinv[:, None]


def _d_body(s_ref, y_ref, deg_ref, b_ref, xo_ref, h_ref):
    n = y_ref.shape[0]
    deg = jnp.sum(deg_ref[...], axis=0) + 1.0
    dinv = lax.rsqrt(deg)
    tot = (s_ref[0, :n, :] + s_ref[1, :n, :] + y_ref[...]) * dinv[:, None] \
        + b_ref[...][None, :]
    xo = jnp.maximum(tot, 0.0)
    xo_ref[...] = xo
    h_ref[...] = jax.nn.sigmoid(jnp.mean(xo, axis=0, keepdims=True))


def kernel(feature, edge_index, W, b):
    n, f_in = feature.shape
    f_out = W.shape[1]
    e = edge_index.shape[1]

    info = plsc.get_sparse_core_info()
    nc, ns = info.num_cores, info.num_subcores
    nw = nc * ns
    epw = e // nw
    k = 125
    ch = epw // k
    align = ns * 128
    n_pad = (n + align - 1) // align * align
    assert epw * nw == e and ch * k == epw and ch % 16 == 0
    assert f_out % _LANES == 0

    src3 = edge_index[0].reshape(nw, ch, k)
    dst3 = edge_index[1].reshape(nw, ch, k)
    nit = epw // _LANES
    dst_h = edge_index[1].reshape(nw, nit, _LANES)

    deg_parts = _build_hist(n, nw, ns, nit)(dst_h)

    y = pl.pallas_call(
        _b_body,
        out_shape=jax.ShapeDtypeStruct((n, f_out), jnp.float32),
    )(deg_parts, feature, W)

    s_parts = _build_scatter(n_pad, f_out, nc, ns, ch, k)(y, src3, dst3)

    x_out, h = pl.pallas_call(
        _d_body,
        out_shape=(
            jax.ShapeDtypeStruct((n, f_out), jnp.float32),
            jax.ShapeDtypeStruct((1, f_out), jnp.float32),
        ),
    )(s_parts, y, deg_parts, b)
    return (x_out, h.reshape(f_out))


# speedup vs baseline: 1.0528x; 1.0224x over previous
import functools

import jax
import jax.numpy as jnp
from jax import lax
from jax.experimental import pallas as pl
from jax.experimental.pallas import tpu as pltpu
from jax.experimental.pallas import tpu_sc as plsc

_LANES = 16


def _sc_mesh():
    return plsc.VectorSubcoreMesh(core_axis_name="c", subcore_axis_name="s")


def _build_hist(n, nw, ns, nit):
    @functools.partial(
        pl.kernel,
        out_type=jax.ShapeDtypeStruct((nw, n), jnp.float32),
        mesh=_sc_mesh(),
        scratch_types=[
            pltpu.VMEM((nit, _LANES), jnp.int32),
            pltpu.VMEM((n,), jnp.float32),
        ],
        compiler_params=pltpu.CompilerParams(needs_layout_passes=False),
    )
    def hist(dst_hbm, out_hbm, idx_v, hist_v):
        cid = lax.axis_index("c")
        sid = lax.axis_index("s")
        wid = cid * ns + sid
        pltpu.sync_copy(dst_hbm.at[wid], idx_v)

        zero16 = jnp.zeros((_LANES,), jnp.float32)

        def zbody(i, carry):
            hist_v[pl.ds(i * _LANES, _LANES)] = zero16
            return carry

        lax.fori_loop(0, n // _LANES, zbody, 0)

        ones16 = jnp.ones((_LANES,), jnp.float32)

        def body(i, carry):
            idx = idx_v[i, :]
            plsc.addupdate_scatter(hist_v, [idx], ones16)
            return carry

        lax.fori_loop(0, nit, body, 0)
        pltpu.sync_copy(hist_v, out_hbm.at[wid])

    return hist


def _build_scatter(n_pad, f, nc, ns, ch, k):
    rows_per_tile = n_pad // ns
    zrows = 128
    qch = 16
    nq = ch // qch

    @functools.partial(
        pl.kernel,
        out_type=jax.ShapeDtypeStruct((nc, n_pad, f), jnp.float32),
        mesh=_sc_mesh(),
        scratch_types=[
            pltpu.VMEM((2, qch, k), jnp.int32),
            pltpu.VMEM((2, qch, k), jnp.int32),
            pltpu.VMEM((zrows, f), jnp.float32),
            pltpu.VMEM((k, f), jnp.float32),
            pltpu.VMEM_SHARED((n_pad, f), jnp.float32),
            pltpu.SemaphoreType.DMA,
            pltpu.SemaphoreType.DMA,
            pltpu.SemaphoreType.DMA,
        ],
    )
    def scat(y_hbm, src_hbm, dst_hbm, out_hbm, src_v, dst_v, rows0, rows1,
             agg_sh, sem0, sem1, semi):
        cid = lax.axis_index("c")
        sid = lax.axis_index("s")
        wid = cid * ns + sid

        zero16 = jnp.zeros((_LANES,), jnp.float32)
        fparts = f // _LANES

        def zbody(i, carry):
            r = i // fparts
            c = i % fparts
            rows0[r, pl.ds(c * _LANES, _LANES)] = zero16
            return carry

        lax.fori_loop(0, zrows * fparts, zbody, 0)

        base = sid * rows_per_tile
        for j in range(rows_per_tile // zrows):
            pltpu.sync_copy(rows0, agg_sh.at[pl.ds(base + j * zrows, zrows)])
        plsc.subcore_barrier()

        rows0k = rows0.at[pl.ds(0, k)]

        pltpu.sync_copy(src_hbm.at[wid, pl.ds(0, qch)], src_v.at[0])
        pltpu.sync_copy(dst_hbm.at[wid, pl.ds(0, qch)], dst_v.at[0])
        for q in range(nq):
            p = q % 2
            sq = src_v.at[p]
            dq = dst_v.at[p]
            if q > 0:
                pltpu.make_async_copy(
                    src_hbm.at[wid, pl.ds(q * qch, qch)], sq, semi).wait()
                pltpu.make_async_copy(
                    dst_hbm.at[wid, pl.ds(q * qch, qch)], dq, semi).wait()
            if q + 1 < nq:
                pn = (q + 1) % 2
                pltpu.async_copy(
                    src_hbm.at[wid, pl.ds((q + 1) * qch, qch)],
                    src_v.at[pn], semi)
                pltpu.async_copy(
                    dst_hbm.at[wid, pl.ds((q + 1) * qch, qch)],
                    dst_v.at[pn], semi)
            pltpu.async_copy(y_hbm.at[sq.at[0]], rows0k, sem0)

            def pair(jj, carry):
                j0 = 2 * jj
                pltpu.make_async_copy(y_hbm.at[sq.at[j0]], rows0k,
                                      sem0).wait()
                pltpu.async_copy(y_hbm.at[sq.at[j0 + 1]], rows1, sem1)
                pltpu.sync_copy(rows0k, agg_sh.at[dq.at[j0]], add=True)
                pltpu.make_async_copy(y_hbm.at[sq.at[j0 + 1]], rows1,
                                      sem1).wait()
                pltpu.async_copy(y_hbm.at[sq.at[j0 + 2]], rows0k, sem0)
                pltpu.sync_copy(rows1, agg_sh.at[dq.at[j0 + 1]], add=True)
                return carry

            lax.fori_loop(0, qch // 2 - 1, pair, 0)

            j0 = qch - 2
            pltpu.make_async_copy(y_hbm.at[sq.at[j0]], rows0k, sem0).wait()
            pltpu.async_copy(y_hbm.at[sq.at[j0 + 1]], rows1, sem1)
            pltpu.sync_copy(rows0k, agg_sh.at[dq.at[j0]], add=True)
            pltpu.make_async_copy(y_hbm.at[sq.at[j0 + 1]], rows1,
                                  sem1).wait()
            pltpu.sync_copy(rows1, agg_sh.at[dq.at[j0 + 1]], add=True)

        plsc.subcore_barrier()
        pltpu.sync_copy(
            agg_sh.at[pl.ds(base, rows_per_tile)],
            out_hbm.at[cid, pl.ds(base, rows_per_tile)],
        )

    return scat


def _b_body(deg_ref, f_ref, w_ref, y_ref):
    deg = jnp.sum(deg_ref[...], axis=0) + 1.0
    dinv = lax.rsqrt(deg)
    x = jnp.dot(f_ref[...], w_ref[...], preferred_element_type=jnp.float32)
    y_ref[...] = x * dinv[:, None]


def _d_body(s_ref, y_ref, deg_ref, b_ref, xo_ref, h_ref):
    n = y_ref.shape[0]
    deg = jnp.sum(deg_ref[...], axis=0) + 1.0
    dinv = lax.rsqrt(deg)
    tot = (s_ref[0, :n, :] + s_ref[1, :n, :] + y_ref[...]) * dinv[:, None] \
        + b_ref[...][None, :]
    xo = jnp.maximum(tot, 0.0)
    xo_ref[...] = xo
    h_ref[...] = jax.nn.sigmoid(jnp.mean(xo, axis=0, keepdims=True))


def kernel(feature, edge_index, W, b):
    n, f_in = feature.shape
    f_out = W.shape[1]
    e = edge_index.shape[1]

    info = plsc.get_sparse_core_info()
    nc, ns = info.num_cores, info.num_subcores
    nw = nc * ns
    epw = e // nw
    k = 125
    ch = epw // k
    align = ns * 128
    n_pad = (n + align - 1) // align * align
    assert epw * nw == e and ch * k == epw and ch % 16 == 0
    assert f_out % _LANES == 0

    src3 = edge_index[0].reshape(nw, ch, k)
    dst3 = edge_index[1].reshape(nw, ch, k)
    nit = epw // _LANES
    dst_h = edge_index[1].reshape(nw, nit, _LANES)

    deg_parts = _build_hist(n, nw, ns, nit)(dst_h)

    y = pl.pallas_call(
        _b_body,
        out_shape=jax.ShapeDtypeStruct((n, f_out), jnp.float32),
    )(deg_parts, feature, W)

    s_parts = _build_scatter(n_pad, f_out, nc, ns, ch, k)(y, src3, dst3)

    x_out, h = pl.pallas_call(
        _d_body,
        out_shape=(
            jax.ShapeDtypeStruct((n, f_out), jnp.float32),
            jax.ShapeDtypeStruct((1, f_out), jnp.float32),
        ),
    )(s_parts, y, deg_parts, b)
    return (x_out, h.reshape(f_out))


# speedup vs baseline: 1.0664x; 1.0130x over previous
import functools

import jax
import jax.numpy as jnp
from jax import lax
from jax.experimental import pallas as pl
from jax.experimental.pallas import tpu as pltpu
from jax.experimental.pallas import tpu_sc as plsc

_LANES = 16


def _sc_mesh():
    return plsc.VectorSubcoreMesh(core_axis_name="c", subcore_axis_name="s")


def _build_hist(n, nw, ns, nit):
    @functools.partial(
        pl.kernel,
        out_type=jax.ShapeDtypeStruct((nw, n), jnp.float32),
        mesh=_sc_mesh(),
        scratch_types=[
            pltpu.VMEM((nit, _LANES), jnp.int32),
            pltpu.VMEM((n,), jnp.float32),
        ],
        compiler_params=pltpu.CompilerParams(needs_layout_passes=False),
    )
    def hist(dst_hbm, out_hbm, idx_v, hist_v):
        cid = lax.axis_index("c")
        sid = lax.axis_index("s")
        wid = cid * ns + sid
        pltpu.sync_copy(dst_hbm.at[wid], idx_v)

        zero16 = jnp.zeros((_LANES,), jnp.float32)

        def zbody(i, carry):
            hist_v[pl.ds(i * _LANES, _LANES)] = zero16
            return carry

        lax.fori_loop(0, n // _LANES, zbody, 0)

        ones16 = jnp.ones((_LANES,), jnp.float32)

        def body(i, carry):
            idx = idx_v[i, :]
            plsc.addupdate_scatter(hist_v, [idx], ones16)
            return carry

        lax.fori_loop(0, nit, body, 0)
        pltpu.sync_copy(hist_v, out_hbm.at[wid])

    return hist


def _build_scatter(n_pad, f, nc, ns, ch, k):
    rows_per_tile = n_pad // ns
    zrows = 128
    qch = 16
    nq = ch // qch

    @functools.partial(
        pl.kernel,
        out_type=jax.ShapeDtypeStruct((nc, n_pad, f), jnp.float32),
        mesh=_sc_mesh(),
        scratch_types=[
            pltpu.VMEM((2, qch, k), jnp.int32),
            pltpu.VMEM((2, qch, k), jnp.int32),
            pltpu.VMEM((zrows, f), jnp.float32),
            pltpu.VMEM((k, f), jnp.float32),
            pltpu.VMEM_SHARED((n_pad, f), jnp.float32),
            pltpu.SemaphoreType.DMA,
            pltpu.SemaphoreType.DMA,
            pltpu.SemaphoreType.DMA,
        ],
    )
    def scat(y_hbm, src_hbm, dst_hbm, out_hbm, src_v, dst_v, rows0, rows1,
             agg_sh, sem0, sem1, semi):
        cid = lax.axis_index("c")
        sid = lax.axis_index("s")
        wid = cid * ns + sid

        zero16 = jnp.zeros((_LANES,), jnp.float32)
        fparts = f // _LANES

        def zbody(i, carry):
            r = i // fparts
            c = i % fparts
            rows0[r, pl.ds(c * _LANES, _LANES)] = zero16
            return carry

        lax.fori_loop(0, zrows * fparts, zbody, 0)

        base = sid * rows_per_tile
        for j in range(rows_per_tile // zrows):
            pltpu.sync_copy(rows0, agg_sh.at[pl.ds(base + j * zrows, zrows)])
        plsc.subcore_barrier()

        rows0k = rows0.at[pl.ds(0, k)]

        pltpu.sync_copy(src_hbm.at[wid, pl.ds(0, qch)], src_v.at[0])
        pltpu.sync_copy(dst_hbm.at[wid, pl.ds(0, qch)], dst_v.at[0])
        pltpu.async_copy(y_hbm.at[src_v.at[0].at[0]], rows0k, sem0)
        for q in range(nq):
            p = q % 2
            pn = (q + 1) % 2
            sq = src_v.at[p]
            dq = dst_v.at[p]
            if q + 1 < nq:
                pltpu.async_copy(
                    src_hbm.at[wid, pl.ds((q + 1) * qch, qch)],
                    src_v.at[pn], semi)
                pltpu.async_copy(
                    dst_hbm.at[wid, pl.ds((q + 1) * qch, qch)],
                    dst_v.at[pn], semi)

            def pair(jj, carry):
                j0 = 2 * jj
                pltpu.make_async_copy(y_hbm.at[sq.at[j0]], rows0k,
                                      sem0).wait()
                pltpu.async_copy(y_hbm.at[sq.at[j0 + 1]], rows1, sem1)
                pltpu.sync_copy(rows0k, agg_sh.at[dq.at[j0]], add=True)
                pltpu.make_async_copy(y_hbm.at[sq.at[j0 + 1]], rows1,
                                      sem1).wait()
                pltpu.async_copy(y_hbm.at[sq.at[j0 + 2]], rows0k, sem0)
                pltpu.sync_copy(rows1, agg_sh.at[dq.at[j0 + 1]], add=True)
                return carry

            lax.fori_loop(0, qch // 2 - 1, pair, 0)

            j0 = qch - 2
            pltpu.make_async_copy(y_hbm.at[sq.at[j0]], rows0k, sem0).wait()
            pltpu.async_copy(y_hbm.at[sq.at[j0 + 1]], rows1, sem1)
            pltpu.sync_copy(rows0k, agg_sh.at[dq.at[j0]], add=True)
            pltpu.make_async_copy(y_hbm.at[sq.at[j0 + 1]], rows1,
                                  sem1).wait()
            if q + 1 < nq:
                pltpu.make_async_copy(
                    src_hbm.at[wid, pl.ds((q + 1) * qch, qch)],
                    src_v.at[pn], semi).wait()
                pltpu.make_async_copy(
                    dst_hbm.at[wid, pl.ds((q + 1) * qch, qch)],
                    dst_v.at[pn], semi).wait()
                pltpu.async_copy(y_hbm.at[src_v.at[pn].at[0]], rows0k, sem0)
            pltpu.sync_copy(rows1, agg_sh.at[dq.at[j0 + 1]], add=True)

        plsc.subcore_barrier()
        pltpu.sync_copy(
            agg_sh.at[pl.ds(base, rows_per_tile)],
            out_hbm.at[cid, pl.ds(base, rows_per_tile)],
        )

    return scat


def _b_body(deg_ref, f_ref, w_ref, y_ref):
    deg = jnp.sum(deg_ref[...], axis=0) + 1.0
    dinv = lax.rsqrt(deg)
    x = jnp.dot(f_ref[...], w_ref[...], preferred_element_type=jnp.float32)
    y_ref[...] = x * dinv[:, None]


def _d_body(s_ref, y_ref, deg_ref, b_ref, xo_ref, h_ref):
    n = y_ref.shape[0]
    deg = jnp.sum(deg_ref[...], axis=0) + 1.0
    dinv = lax.rsqrt(deg)
    tot = (s_ref[0, :n, :] + s_ref[1, :n, :] + y_ref[...]) * dinv[:, None] \
        + b_ref[...][None, :]
    xo = jnp.maximum(tot, 0.0)
    xo_ref[...] = xo
    h_ref[...] = jax.nn.sigmoid(jnp.mean(xo, axis=0, keepdims=True))


def kernel(feature, edge_index, W, b):
    n, f_in = feature.shape
    f_out = W.shape[1]
    e = edge_index.shape[1]

    info = plsc.get_sparse_core_info()
    nc, ns = info.num_cores, info.num_subcores
    nw = nc * ns
    epw = e // nw
    k = 125
    ch = epw // k
    align = ns * 128
    n_pad = (n + align - 1) // align * align
    assert epw * nw == e and ch * k == epw and ch % 16 == 0
    assert f_out % _LANES == 0

    src3 = edge_index[0].reshape(nw, ch, k)
    dst3 = edge_index[1].reshape(nw, ch, k)
    nit = epw // _LANES
    dst_h = edge_index[1].reshape(nw, nit, _LANES)

    deg_parts = _build_hist(n, nw, ns, nit)(dst_h)

    y = pl.pallas_call(
        _b_body,
        out_shape=jax.ShapeDtypeStruct((n, f_out), jnp.float32),
    )(deg_parts, feature, W)

    s_parts = _build_scatter(n_pad, f_out, nc, ns, ch, k)(y, src3, dst3)

    x_out, h = pl.pallas_call(
        _d_body,
        out_shape=(
            jax.ShapeDtypeStruct((n, f_out), jnp.float32),
            jax.ShapeDtypeStruct((1, f_out), jnp.float32),
        ),
    )(s_parts, y, deg_parts, b)
    return (x_out, h.reshape(f_out))


# speedup vs baseline: 1.0700x; 1.0033x over previous
import functools

import jax
import jax.numpy as jnp
from jax import lax
from jax.experimental import pallas as pl
from jax.experimental.pallas import tpu as pltpu
from jax.experimental.pallas import tpu_sc as plsc

_LANES = 16


def _sc_mesh():
    return plsc.VectorSubcoreMesh(core_axis_name="c", subcore_axis_name="s")


def _build_hist(n, nw, ns, nit):
    @functools.partial(
        pl.kernel,
        out_type=jax.ShapeDtypeStruct((nw, n), jnp.float32),
        mesh=_sc_mesh(),
        scratch_types=[
            pltpu.VMEM((nit, _LANES), jnp.int32),
            pltpu.VMEM((n,), jnp.float32),
        ],
        compiler_params=pltpu.CompilerParams(needs_layout_passes=False),
    )
    def hist(dst_hbm, out_hbm, idx_v, hist_v):
        cid = lax.axis_index("c")
        sid = lax.axis_index("s")
        wid = cid * ns + sid
        pltpu.sync_copy(dst_hbm.at[wid], idx_v)

        zero16 = jnp.zeros((_LANES,), jnp.float32)

        def zbody(i, carry):
            hist_v[pl.ds(i * _LANES, _LANES)] = zero16
            return carry

        lax.fori_loop(0, n // _LANES, zbody, 0)

        ones16 = jnp.ones((_LANES,), jnp.float32)

        def body(i, carry):
            idx = idx_v[i, :]
            plsc.addupdate_scatter(hist_v, [idx], ones16)
            return carry

        lax.fori_loop(0, nit, body, 0)
        pltpu.sync_copy(hist_v, out_hbm.at[wid])

    return hist


def _build_scatter(n_pad, f, nc, ns, ch, k):
    rows_per_tile = n_pad // ns
    zrows = 128
    qch = 16
    nq = ch // qch

    @functools.partial(
        pl.kernel,
        out_type=jax.ShapeDtypeStruct((nc, n_pad, f), jnp.float32),
        mesh=_sc_mesh(),
        scratch_types=[
            pltpu.VMEM((2, qch, k), jnp.int32),
            pltpu.VMEM((2, qch, k), jnp.int32),
            pltpu.VMEM((zrows, f), jnp.float32),
            pltpu.VMEM((k, f), jnp.float32),
            pltpu.VMEM_SHARED((n_pad, f), jnp.float32),
            pltpu.SemaphoreType.DMA,
            pltpu.SemaphoreType.DMA,
            pltpu.SemaphoreType.DMA,
        ],
    )
    def scat(y_hbm, src_hbm, dst_hbm, out_hbm, src_v, dst_v, rows0, rows1,
             agg_sh, sem0, sem1, semi):
        cid = lax.axis_index("c")
        sid = lax.axis_index("s")
        wid = cid * ns + sid

        pltpu.async_copy(src_hbm.at[wid, pl.ds(0, qch)], src_v.at[0], semi)
        pltpu.async_copy(dst_hbm.at[wid, pl.ds(0, qch)], dst_v.at[0], semi)

        zero16 = jnp.zeros((_LANES,), jnp.float32)
        fparts = f // _LANES

        def zbody(i, carry):
            r = i // fparts
            c = i % fparts
            rows0[r, pl.ds(c * _LANES, _LANES)] = zero16
            return carry

        lax.fori_loop(0, zrows * fparts, zbody, 0)

        base = sid * rows_per_tile
        for j in range(rows_per_tile // zrows):
            pltpu.async_copy(rows0, agg_sh.at[pl.ds(base + j * zrows, zrows)],
                             sem1)
        pltpu.make_async_copy(src_hbm.at[wid, pl.ds(0, qch)], src_v.at[0],
                              semi).wait()
        pltpu.make_async_copy(dst_hbm.at[wid, pl.ds(0, qch)], dst_v.at[0],
                              semi).wait()
        for j in range(rows_per_tile // zrows):
            pltpu.make_async_copy(rows0,
                                  agg_sh.at[pl.ds(base + j * zrows, zrows)],
                                  sem1).wait()
        plsc.subcore_barrier()

        rows0k = rows0.at[pl.ds(0, k)]

        pltpu.async_copy(y_hbm.at[src_v.at[0].at[0]], rows0k, sem0)
        for q in range(nq):
            p = q % 2
            pn = (q + 1) % 2
            sq = src_v.at[p]
            dq = dst_v.at[p]
            if q + 1 < nq:
                pltpu.async_copy(
                    src_hbm.at[wid, pl.ds((q + 1) * qch, qch)],
                    src_v.at[pn], semi)
                pltpu.async_copy(
                    dst_hbm.at[wid, pl.ds((q + 1) * qch, qch)],
                    dst_v.at[pn], semi)

            def pair(jj, carry):
                j0 = 2 * jj
                pltpu.make_async_copy(y_hbm.at[sq.at[j0]], rows0k,
                                      sem0).wait()
                pltpu.async_copy(y_hbm.at[sq.at[j0 + 1]], rows1, sem1)
                pltpu.sync_copy(rows0k, agg_sh.at[dq.at[j0]], add=True)
                pltpu.make_async_copy(y_hbm.at[sq.at[j0 + 1]], rows1,
                                      sem1).wait()
                pltpu.async_copy(y_hbm.at[sq.at[j0 + 2]], rows0k, sem0)
                pltpu.sync_copy(rows1, agg_sh.at[dq.at[j0 + 1]], add=True)
                return carry

            lax.fori_loop(0, qch // 2 - 1, pair, 0)

            j0 = qch - 2
            pltpu.make_async_copy(y_hbm.at[sq.at[j0]], rows0k, sem0).wait()
            pltpu.async_copy(y_hbm.at[sq.at[j0 + 1]], rows1, sem1)
            pltpu.sync_copy(rows0k, agg_sh.at[dq.at[j0]], add=True)
            pltpu.make_async_copy(y_hbm.at[sq.at[j0 + 1]], rows1,
                                  sem1).wait()
            if q + 1 < nq:
                pltpu.make_async_copy(
                    src_hbm.at[wid, pl.ds((q + 1) * qch, qch)],
                    src_v.at[pn], semi).wait()
                pltpu.make_async_copy(
                    dst_hbm.at[wid, pl.ds((q + 1) * qch, qch)],
                    dst_v.at[pn], semi).wait()
                pltpu.async_copy(y_hbm.at[src_v.at[pn].at[0]], rows0k, sem0)
            pltpu.sync_copy(rows1, agg_sh.at[dq.at[j0 + 1]], add=True)

        plsc.subcore_barrier()
        pltpu.sync_copy(
            agg_sh.at[pl.ds(base, rows_per_tile)],
            out_hbm.at[cid, pl.ds(base, rows_per_tile)],
        )

    return scat


def _b_body(deg_ref, f_ref, w_ref, y_ref):
    deg = jnp.sum(deg_ref[...], axis=0) + 1.0
    dinv = lax.rsqrt(deg)
    x = jnp.dot(f_ref[...], w_ref[...], preferred_element_type=jnp.float32)
    y_ref[...] = x * dinv[:, None]


def _d_body(s_ref, y_ref, deg_ref, b_ref, xo_ref, h_ref):
    n = y_ref.shape[0]
    deg = jnp.sum(deg_ref[...], axis=0) + 1.0
    dinv = lax.rsqrt(deg)
    tot = (s_ref[0, :n, :] + s_ref[1, :n, :] + y_ref[...]) * dinv[:, None] \
        + b_ref[...][None, :]
    xo = jnp.maximum(tot, 0.0)
    xo_ref[...] = xo
    h_ref[...] = jax.nn.sigmoid(jnp.mean(xo, axis=0, keepdims=True))


def kernel(feature, edge_index, W, b):
    n, f_in = feature.shape
    f_out = W.shape[1]
    e = edge_index.shape[1]

    info = plsc.get_sparse_core_info()
    nc, ns = info.num_cores, info.num_subcores
    nw = nc * ns
    epw = e // nw
    k = 125
    ch = epw // k
    align = ns * 128
    n_pad = (n + align - 1) // align * align
    assert epw * nw == e and ch * k == epw and ch % 16 == 0
    assert f_out % _LANES == 0

    src3 = edge_index[0].reshape(nw, ch, k)
    dst3 = edge_index[1].reshape(nw, ch, k)
    nit = epw // _LANES
    dst_h = edge_index[1].reshape(nw, nit, _LANES)

    deg_parts = _build_hist(n, nw, ns, nit)(dst_h)

    y = pl.pallas_call(
        _b_body,
        out_shape=jax.ShapeDtypeStruct((n, f_out), jnp.float32),
    )(deg_parts, feature, W)

    s_parts = _build_scatter(n_pad, f_out, nc, ns, ch, k)(y, src3, dst3)

    x_out, h = pl.pallas_call(
        _d_body,
        out_shape=(
            jax.ShapeDtypeStruct((n, f_out), jnp.float32),
            jax.ShapeDtypeStruct((1, f_out), jnp.float32),
        ),
    )(s_parts, y, deg_parts, b)
    return (x_out, h.reshape(f_out))
